# baseline (speedup 1.0000x reference)
import jax
import jax.numpy as jnp
from jax.experimental import pallas as pl
from jax.experimental.pallas import tpu as pltpu

_B, _S, _D_MODEL, _D_SPACE = 4, 2048, 4096, 64
_N_POOL = 512
_HALF = _N_POOL // 2
_RV_END = _N_POOL * 6
_TOPKS = (8, 8, 3, 8, 8, 3)
_TILE = 256
_STRIP = 64


def _router_kernel(x_ref, w_ref, b_ref, emb_ref, out_ref, emb_n_ref):
    @pl.when(pl.program_id(0) == 0)
    def _():
        emb = emb_ref[...]
        inv = 1.0 / jnp.maximum(
            jnp.sqrt(jnp.sum(emb * emb, axis=1, keepdims=True)), 1e-12
        )
        emb_n_ref[...] = emb * inv

    x = x_ref[...]
    w = w_ref[...]
    proj = jax.lax.dot_general(
        x, w, (((1,), (0,)), ((), ())), preferred_element_type=jnp.float32
    )
    proj = proj + b_ref[...]
    emb_n = emb_n_ref[...]
    neg = jnp.float32(-jnp.inf)
    for g in range(6):
        e = emb_n[g * _N_POOL:(g + 1) * _N_POOL, :]
        k = _TOPKS[g]
        for s in range(0, _TILE, _STRIP):
            h = proj[s:s + _STRIP, g * _D_SPACE:(g + 1) * _D_SPACE]
            logits = jax.lax.dot_general(
                h, e, (((1,), (1,)), ((), ())), preferred_element_type=jnp.float32
            )
            c = jnp.maximum(logits[:, :_HALF], logits[:, _HALF:])
            n = jnp.minimum(logits[:, :_HALF], logits[:, _HALF:])
            t = jnp.max(c, axis=1, keepdims=True)
            m = t
            for _ in range(k - 1):
                hit = c >= t
                c = jnp.where(hit, n, c)
                n = jnp.where(hit, neg, n)
                t = jnp.max(c, axis=1, keepdims=True)
            ex = jnp.exp(logits - m)
            rz = 1.0 / jnp.sum(ex, axis=1, keepdims=True)
            out_ref[s:s + _STRIP, g * _N_POOL:(g + 1) * _N_POOL] = jnp.where(
                logits >= t, ex * rz, 0.0
            )


def kernel(x, importance, W_proj, b_proj, neuron_emb):
    del importance
    xf = x.reshape(_B * _S, _D_MODEL)
    emb = neuron_emb[:_RV_END]
    b2 = b_proj.reshape(1, _D_SPACE * 6)
    out = pl.pallas_call(
        _router_kernel,
        grid=(_B * _S // _TILE,),
        in_specs=[
            pl.BlockSpec((_TILE, _D_MODEL), lambda i: (i, 0)),
            pl.BlockSpec((_D_MODEL, _D_SPACE * 6), lambda i: (0, 0)),
            pl.BlockSpec((1, _D_SPACE * 6), lambda i: (0, 0)),
            pl.BlockSpec((_RV_END, _D_SPACE), lambda i: (0, 0)),
        ],
        out_specs=pl.BlockSpec((_TILE, _RV_END), lambda i: (i, 0)),
        out_shape=jax.ShapeDtypeStruct((_B * _S, _RV_END), jnp.float32),
        scratch_shapes=[pltpu.VMEM((_RV_END, _D_SPACE), jnp.float32)],
        compiler_params=pltpu.CompilerParams(dimension_semantics=("arbitrary",)),
    )(xf, W_proj, b2, emb)
    return out.reshape(_B, _S, _RV_END)


# speedup vs baseline: 1.0423x; 1.0423x over previous
import jax
import jax.numpy as jnp
from jax.experimental import pallas as pl
from jax.experimental.pallas import tpu as pltpu

_B, _S, _D_MODEL, _D_SPACE = 4, 2048, 4096, 64
_N_POOL = 512
_Q = _N_POOL // 4
_RV_END = _N_POOL * 6
_TOPKS = (8, 8, 3, 8, 8, 3)
_TILE = 512
_STRIP = 64


def _router_kernel(x_ref, w_ref, b_ref, emb_ref, out_ref):
    x = x_ref[...]
    w = w_ref[...]
    proj = jax.lax.dot_general(
        x, w, (((1,), (0,)), ((), ())), preferred_element_type=jnp.float32
    )
    proj = proj + b_ref[...]
    emb = emb_ref[...]
    inv = 1.0 / jnp.maximum(
        jnp.sqrt(jnp.sum(emb * emb, axis=1, keepdims=True)), 1e-12
    )
    emb_n = emb * inv
    neg = jnp.float32(-jnp.inf)
    for g in range(6):
        e = emb_n[g * _N_POOL:(g + 1) * _N_POOL, :]
        k = _TOPKS[g]
        for s in range(0, _TILE, _STRIP):
            h = proj[s:s + _STRIP, g * _D_SPACE:(g + 1) * _D_SPACE]
            logits = jax.lax.dot_general(
                h, e, (((1,), (1,)), ((), ())), preferred_element_type=jnp.float32
            )
            c = jnp.maximum(logits[:, :2 * _Q], logits[:, 2 * _Q:])
            n = jnp.minimum(logits[:, :2 * _Q], logits[:, 2 * _Q:])
            t = jnp.max(c, axis=1, keepdims=True)
            m = t
            for _ in range(k - 1):
                hit = c >= t
                c = jnp.where(hit, n, c)
                n = jnp.where(hit, neg, n)
                t = jnp.max(c, axis=1, keepdims=True)
            ex = jnp.exp(logits - m)
            rz = 1.0 / jnp.sum(ex, axis=1, keepdims=True)
            out_ref[s:s + _STRIP, g * _N_POOL:(g + 1) * _N_POOL] = jnp.where(
                logits >= t, ex * rz, 0.0
            )


def kernel(x, importance, W_proj, b_proj, neuron_emb):
    del importance
    xf = x.reshape(_B * _S, _D_MODEL)
    emb = neuron_emb[:_RV_END]
    b2 = b_proj.reshape(1, _D_SPACE * 6)
    out = pl.pallas_call(
        _router_kernel,
        grid=(_B * _S // _TILE,),
        in_specs=[
            pl.BlockSpec((_TILE, _D_MODEL), lambda i: (i, 0)),
            pl.BlockSpec((_D_MODEL, _D_SPACE * 6), lambda i: (0, 0)),
            pl.BlockSpec((1, _D_SPACE * 6), lambda i: (0, 0)),
            pl.BlockSpec((_RV_END, _D_SPACE), lambda i: (0, 0)),
        ],
        out_specs=pl.BlockSpec((_TILE, _RV_END), lambda i: (i, 0)),
        out_shape=jax.ShapeDtypeStruct((_B * _S, _RV_END), jnp.float32),
        compiler_params=pltpu.CompilerParams(dimension_semantics=("arbitrary",)),
    )(xf, W_proj, b2, emb)
    return out.reshape(_B, _S, _RV_END)


# speedup vs baseline: 1.0658x; 1.0226x over previous
import jax
import jax.numpy as jnp
from jax.experimental import pallas as pl
from jax.experimental.pallas import tpu as pltpu

_B, _S, _D_MODEL, _D_SPACE = 4, 2048, 4096, 64
_N_POOL = 512
_RV_END = _N_POOL * 6
_TOPKS = (8, 8, 3, 8, 8, 3)
_TILE = 256


def _router_kernel(x_ref, w_ref, b_ref, emb_ref, out_ref):
    x = x_ref[...]
    w = w_ref[...]
    proj = jax.lax.dot_general(
        x, w, (((1,), (0,)), ((), ())), preferred_element_type=jnp.float32
    )
    proj = proj + b_ref[...]
    emb = emb_ref[...]
    inv_norm = 1.0 / jnp.maximum(
        jnp.sqrt(jnp.sum(emb * emb, axis=1, keepdims=True)), 1e-12
    )
    emb_n = emb * inv_norm
    neg = jnp.float32(-jnp.inf)
    for g in range(6):
        h = proj[:, g * _D_SPACE:(g + 1) * _D_SPACE]
        e = emb_n[g * _N_POOL:(g + 1) * _N_POOL, :]
        k = _TOPKS[g]
        logits = jax.lax.dot_general(
            h, e, (((1,), (1,)), ((), ())), preferred_element_type=jnp.float32
        )
        t = jnp.max(logits, axis=1, keepdims=True)
        m = t
        vals = logits
        for j in range(k - 1):
            vals = jnp.where(vals >= t, neg, vals)
            t = jnp.max(vals, axis=1, keepdims=True)
        ex = jnp.exp(logits - m)
        rz = 1.0 / jnp.sum(ex, axis=1, keepdims=True)
        out_ref[:, g * _N_POOL:(g + 1) * _N_POOL] = jnp.where(
            logits >= t, ex * rz, 0.0
        )


def kernel(x, importance, W_proj, b_proj, neuron_emb):
    del importance
    xf = x.reshape(_B * _S, _D_MODEL)
    emb = neuron_emb[:_RV_END]
    b2 = b_proj.reshape(1, _D_SPACE * 6)
    out = pl.pallas_call(
        _router_kernel,
        grid=(_B * _S // _TILE,),
        in_specs=[
            pl.BlockSpec((_TILE, _D_MODEL), lambda i: (i, 0)),
            pl.BlockSpec((_D_MODEL, _D_SPACE * 6), lambda i: (0, 0)),
            pl.BlockSpec((1, _D_SPACE * 6), lambda i: (0, 0)),
            pl.BlockSpec((_RV_END, _D_SPACE), lambda i: (0, 0)),
        ],
        out_specs=pl.BlockSpec((_TILE, _RV_END), lambda i: (i, 0)),
        out_shape=jax.ShapeDtypeStruct((_B * _S, _RV_END), jnp.float32),
        compiler_params=pltpu.CompilerParams(dimension_semantics=("arbitrary",)),
    )(xf, W_proj, b2, emb)
    return out.reshape(_B, _S, _RV_END)


# speedup vs baseline: 1.1170x; 1.0480x over previous
import jax
import jax.numpy as jnp
from jax.experimental import pallas as pl
from jax.experimental.pallas import tpu as pltpu

_B, _S, _D_MODEL, _D_SPACE = 4, 2048, 4096, 64
_N_POOL = 512
_RV_END = _N_POOL * 6
_TOPKS = (8, 8, 3, 8, 8, 3)
_TILE = 512


def _router_kernel(x_ref, w_ref, b_ref, emb_ref, out_ref):
    x = x_ref[...]
    w = w_ref[...]
    proj = jax.lax.dot_general(
        x, w, (((1,), (0,)), ((), ())), preferred_element_type=jnp.float32
    )
    proj = proj + b_ref[...]
    emb = emb_ref[...]
    inv_norm = 1.0 / jnp.maximum(
        jnp.sqrt(jnp.sum(emb * emb, axis=1, keepdims=True)), 1e-12
    )
    emb_n = emb * inv_norm
    neg = jnp.float32(-jnp.inf)
    for g in range(6):
        h = proj[:, g * _D_SPACE:(g + 1) * _D_SPACE]
        e = emb_n[g * _N_POOL:(g + 1) * _N_POOL, :]
        k = _TOPKS[g]
        logits = jax.lax.dot_general(
            h, e, (((1,), (1,)), ((), ())), preferred_element_type=jnp.float32
        )
        t = jnp.max(logits, axis=1, keepdims=True)
        m = t
        vals = logits
        for j in range(k - 1):
            vals = jnp.where(vals >= t, neg, vals)
            t = jnp.max(vals, axis=1, keepdims=True)
        ex = jnp.exp(logits - m)
        rz = 1.0 / jnp.sum(ex, axis=1, keepdims=True)
        out_ref[:, g * _N_POOL:(g + 1) * _N_POOL] = jnp.where(
            logits >= t, ex * rz, 0.0
        )


def kernel(x, importance, W_proj, b_proj, neuron_emb):
    del importance
    xf = x.reshape(_B * _S, _D_MODEL)
    emb = neuron_emb[:_RV_END]
    b2 = b_proj.reshape(1, _D_SPACE * 6)
    out = pl.pallas_call(
        _router_kernel,
        grid=(_B * _S // _TILE,),
        in_specs=[
            pl.BlockSpec((_TILE, _D_MODEL), lambda i: (i, 0)),
            pl.BlockSpec((_D_MODEL, _D_SPACE * 6), lambda i: (0, 0)),
            pl.BlockSpec((1, _D_SPACE * 6), lambda i: (0, 0)),
            pl.BlockSpec((_RV_END, _D_SPACE), lambda i: (0, 0)),
        ],
        out_specs=pl.BlockSpec((_TILE, _RV_END), lambda i: (i, 0)),
        out_shape=jax.ShapeDtypeStruct((_B * _S, _RV_END), jnp.float32),
        compiler_params=pltpu.CompilerParams(dimension_semantics=("arbitrary",)),
    )(xf, W_proj, b2, emb)
    return out.reshape(_B, _S, _RV_END)
